# baseline (speedup 1.0000x reference)
import functools
import jax
import jax.numpy as jnp
from jax import lax
from jax.experimental import pallas as pl
from jax.experimental.pallas import tpu as pltpu

_SA_CFG = [(1024, 0.1, 32), (256, 0.2, 32), (64, 0.4, 32), (16, 0.8, 32)]

_INTERPRET = False


def _pad_cols(x, d):
    c = x.shape[-1]
    if c == d:
        return x
    return jnp.concatenate(
        [x, jnp.zeros(x.shape[:-1] + (d - c,), x.dtype)], axis=-1)


def _fps_body(S, N, xref, oref):
    X = xref[0]
    Y = xref[1]
    Z = xref[2]
    B = X.shape[0]
    iota = lax.broadcasted_iota(jnp.int32, (B, N), 1)
    iota_s = lax.broadcasted_iota(jnp.int32, (B, S), 1)

    def step(i, carry):
        dist, far, ax, ay, az = carry
        oh = (iota == far).astype(jnp.float32)
        cx = jnp.sum(X * oh, axis=-1, keepdims=True)
        cy = jnp.sum(Y * oh, axis=-1, keepdims=True)
        cz = jnp.sum(Z * oh, axis=-1, keepdims=True)
        slot = (iota_s == i).astype(jnp.float32)
        ax = ax + cx * slot
        ay = ay + cy * slot
        az = az + cz * slot
        dx = X - cx
        dy = Y - cy
        dz = Z - cz
        d = dx * dx + dy * dy + dz * dz
        dist = jnp.minimum(dist, d)
        m = jnp.max(dist, axis=-1, keepdims=True)
        far = jnp.min(jnp.where(dist == m, iota, N), axis=-1, keepdims=True)
        return dist, far, ax, ay, az

    init = (jnp.full((B, N), 1e10, jnp.float32),
            jnp.zeros((B, 1), jnp.int32),
            jnp.zeros((B, S), jnp.float32),
            jnp.zeros((B, S), jnp.float32),
            jnp.zeros((B, S), jnp.float32))
    _, _, ax, ay, az = lax.fori_loop(0, S, step, init)
    oref[0] = ax
    oref[1] = ay
    oref[2] = az


def _fps(xyzc, S):
    three, B, N = xyzc.shape
    fn = functools.partial(_fps_body, S, N)
    return pl.pallas_call(
        fn,
        grid=(1,),
        in_specs=[pl.BlockSpec((3, B, N), lambda i: (0, 0, 0))],
        out_specs=pl.BlockSpec((3, B, S), lambda i: (0, 0, 0)),
        out_shape=jax.ShapeDtypeStruct((3, B, S), jnp.float32),
        interpret=_INTERPRET,
    )(xyzc)


def _bq_body(N, r2, nsample, xref, cref, oref):
    x = xref[0]
    x3 = x[0:3, :]
    c = cref[0]
    TS = c.shape[0]
    aa = jnp.sum(c * c, axis=-1, keepdims=True)
    bb = jnp.sum(x3 * x3, axis=0, keepdims=True)
    ab = lax.dot_general(c, x3, (((1,), (0,)), ((), ())),
                         preferred_element_type=jnp.float32)
    sq = aa + bb - 2.0 * ab
    iota = lax.broadcasted_iota(jnp.int32, (TS, N), 1)
    midx = jnp.where(sq <= r2, iota, N)
    cols = []
    for _ in range(nsample):
        cur = jnp.min(midx, axis=-1, keepdims=True)
        cols.append(cur)
        midx = jnp.where(midx == cur, N, midx)
    g = jnp.concatenate(cols, axis=-1)
    g = jnp.where(g == N, cols[0], g)
    b = pl.program_id(0)
    oref[0] = g + b * N


def _ball_query(xyzT, new_xyz, radius, nsample):
    B, _, N = xyzT.shape
    S = new_xyz.shape[1]
    TS = min(S, 256)
    fn = functools.partial(_bq_body, N, radius * radius, nsample)
    return pl.pallas_call(
        fn,
        grid=(B, S // TS),
        in_specs=[
            pl.BlockSpec((1, 8, N), lambda b, s: (b, 0, 0)),
            pl.BlockSpec((1, TS, 3), lambda b, s: (b, s, 0)),
        ],
        out_specs=pl.BlockSpec((1, TS, nsample), lambda b, s: (b, s, 0)),
        out_shape=jax.ShapeDtypeStruct((B, S, nsample), jnp.int32),
        interpret=_INTERPRET,
    )(xyzT, new_xyz)


def _gather_rows(table, idx):
    if _INTERPRET:
        return jnp.take(table, idx, axis=0)
    from jax.experimental.pallas import tpu_sc as plsc
    V, D = table.shape
    Btot = idx.shape[0]
    info = plsc.get_sparse_core_info()
    NC, NS = info.num_cores, info.num_subcores
    NW = NC * NS
    R = Btot // NW
    chunk = R
    while chunk * D * 4 > 131072:
        chunk //= 2
    n_chunks = R // chunk
    mesh = plsc.VectorSubcoreMesh(core_axis_name="c", subcore_axis_name="s")

    @functools.partial(
        pl.kernel, mesh=mesh,
        out_type=jax.ShapeDtypeStruct((Btot, D), jnp.float32),
        scratch_types=[
            pltpu.VMEM((chunk,), jnp.int32),
            pltpu.VMEM((chunk,), jnp.int32),
            pltpu.VMEM((chunk, D), jnp.float32),
            pltpu.VMEM((chunk, D), jnp.float32),
            pltpu.SemaphoreType.DMA,
            pltpu.SemaphoreType.DMA,
            pltpu.SemaphoreType.DMA,
            pltpu.SemaphoreType.DMA,
        ],
    )
    def k(table_hbm, idx_hbm, out_hbm, i0, i1, r0, r1, g0, g1, s0, s1):
        wid = lax.axis_index("s") * NC + lax.axis_index("c")
        base = wid * R
        idx_v = [i0, i1]
        rows_v = [r0, r1]
        gsem = [g0, g1]
        ssem = [s0, s1]
        gathers = [None, None]
        stores = [None, None]
        pltpu.sync_copy(idx_hbm.at[pl.ds(base, chunk)], idx_v[0])
        gathers[0] = pltpu.async_copy(table_hbm.at[idx_v[0]], rows_v[0],
                                      gsem[0])
        for ci in range(n_chunks):
            buf = ci % 2
            nbuf_i = 1 - buf
            if ci + 1 < n_chunks:
                if stores[nbuf_i] is not None:
                    stores[nbuf_i].wait()
                    stores[nbuf_i] = None
                off_n = base + (ci + 1) * chunk
                pltpu.sync_copy(idx_hbm.at[pl.ds(off_n, chunk)],
                                idx_v[nbuf_i])
                gathers[nbuf_i] = pltpu.async_copy(
                    table_hbm.at[idx_v[nbuf_i]], rows_v[nbuf_i],
                    gsem[nbuf_i])
            gathers[buf].wait()
            off = base + ci * chunk
            stores[buf] = pltpu.async_copy(rows_v[buf],
                                           out_hbm.at[pl.ds(off, chunk)],
                                           ssem[buf])
        for st in stores:
            if st is not None:
                st.wait()

    return k(table, idx)


def _sa_mlp_body(K, n_layers, gref, cref, *args):
    wrefs = args[:2 * n_layers]
    oref = args[2 * n_layers]
    g = gref[...]
    c = cref[...]
    TM = c.shape[0]
    Dp = g.shape[-1]
    delta = jnp.concatenate(
        [c, jnp.zeros((TM, Dp - 3), jnp.float32)], axis=-1)
    g = g - delta[None, :, :]
    x = g.reshape(K * TM, Dp)
    for li in range(n_layers):
        W = wrefs[2 * li][...]
        b = wrefs[2 * li + 1][...]
        x = jnp.dot(x, W, preferred_element_type=jnp.float32) + b
        x = jnp.maximum(x, 0.0)
    h = x.reshape(K, TM, x.shape[-1])
    oref[...] = jnp.max(h, axis=0)


def _sa_mlp(grouped, centers, layers):
    K, M, Dp = grouped.shape
    TM = min(M, 256)
    n_layers = len(layers)
    Dout = layers[-1][0].shape[1]
    wargs = []
    in_specs = [
        pl.BlockSpec((K, TM, Dp), lambda i: (0, i, 0)),
        pl.BlockSpec((TM, 3), lambda i: (i, 0)),
    ]
    for (W, b) in layers:
        if W.shape[0] != Dp:
            Wp = jnp.concatenate(
                [W, jnp.zeros((Dp - W.shape[0], W.shape[1]), W.dtype)], 0)
        else:
            Wp = W
        wargs += [Wp, b.reshape(1, -1)]
        in_specs += [
            pl.BlockSpec(Wp.shape, lambda i: (0, 0)),
            pl.BlockSpec((1, b.shape[0]), lambda i: (0, 0)),
        ]
        Dp = W.shape[1]
    fn = functools.partial(_sa_mlp_body, K, n_layers)
    return pl.pallas_call(
        fn,
        grid=(M // TM,),
        in_specs=in_specs,
        out_specs=pl.BlockSpec((TM, Dout), lambda i: (i, 0)),
        out_shape=jax.ShapeDtypeStruct((M, Dout), jnp.float32),
        interpret=_INTERPRET,
    )(grouped, centers, *wargs)


def _nn3_body(N2, xref, cref, iref, wref):
    x = xref[0]
    x3 = x[0:3, :]
    c = cref[0]
    TS = c.shape[0]
    aa = jnp.sum(c * c, axis=-1, keepdims=True)
    bb = jnp.sum(x3 * x3, axis=0, keepdims=True)
    ab = lax.dot_general(c, x3, (((1,), (0,)), ((), ())),
                         preferred_element_type=jnp.float32)
    sq = aa + bb - 2.0 * ab
    iota = lax.broadcasted_iota(jnp.int32, (TS, N2), 1)
    icols, dcols = [], []
    s = sq
    for _ in range(3):
        d = jnp.min(s, axis=-1, keepdims=True)
        i = jnp.min(jnp.where(s == d, iota, N2), axis=-1, keepdims=True)
        s = jnp.where(iota == i, jnp.float32(3.4e38), s)
        icols.append(i)
        dcols.append(d)
    dist = jnp.maximum(jnp.concatenate(dcols, axis=-1), 0.0)
    recip = 1.0 / (dist + 1e-8)
    w = recip / jnp.sum(recip, axis=-1, keepdims=True)
    b = pl.program_id(0)
    iref[0] = jnp.concatenate(icols, axis=-1) + b * N2
    wref[0] = w


def _nn3(xyz1, xyz2T):
    B, N1, _ = xyz1.shape
    N2 = xyz2T.shape[2]
    TS = min(N1, 256)
    fn = functools.partial(_nn3_body, N2)
    return pl.pallas_call(
        fn,
        grid=(B, N1 // TS),
        in_specs=[
            pl.BlockSpec((1, 8, N2), lambda b, s: (b, 0, 0)),
            pl.BlockSpec((1, TS, 3), lambda b, s: (b, s, 0)),
        ],
        out_specs=[
            pl.BlockSpec((1, TS, 3), lambda b, s: (b, s, 0)),
            pl.BlockSpec((1, TS, 3), lambda b, s: (b, s, 0)),
        ],
        out_shape=[
            jax.ShapeDtypeStruct((B, N1, 3), jnp.int32),
            jax.ShapeDtypeStruct((B, N1, 3), jnp.float32),
        ],
        interpret=_INTERPRET,
    )(xyz2T, xyz1)


def _fp_mlp_body(n_layers, gref, wref, fref, *args):
    wrefs = args[:2 * n_layers]
    oref = args[2 * n_layers]
    g = gref[...]
    w = wref[...]
    f1 = fref[...]
    interp = (g[0] * w[:, 0:1] + g[1] * w[:, 1:2]) + g[2] * w[:, 2:3]
    x = jnp.concatenate([f1, interp], axis=-1)
    for li in range(n_layers):
        W = wrefs[2 * li][...]
        b = wrefs[2 * li + 1][...]
        x = jnp.dot(x, W, preferred_element_type=jnp.float32) + b
        x = jnp.maximum(x, 0.0)
    oref[...] = x


def _fp_mlp(g, w, f1, layers):
    _, M, D2 = g.shape
    D1 = f1.shape[-1]
    TM = min(M, 512)
    n_layers = len(layers)
    Dout = layers[-1][0].shape[1]
    wargs = []
    in_specs = [
        pl.BlockSpec((3, TM, D2), lambda i: (0, i, 0)),
        pl.BlockSpec((TM, 3), lambda i: (i, 0)),
        pl.BlockSpec((TM, D1), lambda i: (i, 0)),
    ]
    for (W, b) in layers:
        wargs += [W, b.reshape(1, -1)]
        in_specs += [
            pl.BlockSpec(W.shape, lambda i: (0, 0)),
            pl.BlockSpec((1, b.shape[0]), lambda i: (0, 0)),
        ]
    fn = functools.partial(_fp_mlp_body, n_layers)
    return pl.pallas_call(
        fn,
        grid=(M // TM,),
        in_specs=in_specs,
        out_specs=pl.BlockSpec((TM, Dout), lambda i: (i, 0)),
        out_shape=jax.ShapeDtypeStruct((M, Dout), jnp.float32),
        interpret=_INTERPRET,
    )(g, w, f1, *wargs)


def _sa_level(xyz, feats, npoint, radius, nsample, layers):
    B, N, _ = xyz.shape
    C = feats.shape[-1]
    xyzc = jnp.transpose(xyz, (2, 0, 1))
    newc = _fps(xyzc, npoint)
    new_xyz = jnp.transpose(newc, (1, 2, 0))
    xyzT = jnp.concatenate(
        [jnp.transpose(xyz, (0, 2, 1)),
         jnp.zeros((B, 5, N), jnp.float32)], axis=1)
    gidx = _ball_query(xyzT, new_xyz, radius, nsample)
    Din = 3 + C
    Dp = ((Din + 127) // 128) * 128
    table = _pad_cols(jnp.concatenate([xyz, feats], -1),
                      Dp).reshape(B * N, Dp)
    M = B * npoint
    idx_flat = jnp.transpose(gidx.reshape(M, nsample)).reshape(-1)
    rows = _gather_rows(table, idx_flat)
    grouped = rows.reshape(nsample, M, Dp)
    centers = new_xyz.reshape(M, 3)
    feats_out = _sa_mlp(grouped, centers, layers).reshape(B, npoint, -1)
    return new_xyz, feats_out


def _fp_level(xyz1, xyz2, feats1, feats2, layers):
    B, N1, _ = xyz1.shape
    N2 = xyz2.shape[1]
    D2 = feats2.shape[-1]
    xyz2T = jnp.concatenate(
        [jnp.transpose(xyz2, (0, 2, 1)),
         jnp.zeros((B, 5, N2), jnp.float32)], axis=1)
    idx, w = _nn3(xyz1, xyz2T)
    M = B * N1
    table = feats2.reshape(B * N2, D2)
    idx_flat = jnp.transpose(idx.reshape(M, 3)).reshape(-1)
    rows = _gather_rows(table, idx_flat)
    g = rows.reshape(3, M, D2)
    out = _fp_mlp(g, w.reshape(M, 3), feats1.reshape(M, -1), layers)
    return out.reshape(B, N1, -1)


def kernel(pointcloud, params):
    xyz = pointcloud[..., :3]
    feats = pointcloud[..., 3:]
    l_xyz = [xyz]
    l_feats = [feats]
    for cfg, layers in zip(_SA_CFG, params["sa"]):
        nx, nf = _sa_level(l_xyz[-1], l_feats[-1], cfg[0], cfg[1], cfg[2],
                           layers)
        l_xyz.append(nx)
        l_feats.append(nf)
    for i in range(-1, -5, -1):
        l_feats[i - 1] = _fp_level(l_xyz[i - 1], l_xyz[i], l_feats[i - 1],
                                   l_feats[i], params["fp"][i])
    return l_feats[0]


# speedup vs baseline: 1.0739x; 1.0739x over previous
import functools
import jax
import jax.numpy as jnp
from jax import lax
from jax.experimental import pallas as pl
from jax.experimental.pallas import tpu as pltpu

_SA_CFG = [(1024, 0.1, 32), (256, 0.2, 32), (64, 0.4, 32), (16, 0.8, 32)]


def _pad_cols(x, d):
    c = x.shape[-1]
    if c == d:
        return x
    return jnp.concatenate(
        [x, jnp.zeros(x.shape[:-1] + (d - c,), x.dtype)], axis=-1)


def _fps_body(S, N, xref, oref):
    X = xref[0]
    Y = xref[1]
    Z = xref[2]
    B = X.shape[0]
    iota = lax.broadcasted_iota(jnp.int32, (B, N), 1)
    iota_s = lax.broadcasted_iota(jnp.int32, (B, S), 1)

    def step(i, carry):
        dist, far, ax, ay, az = carry
        oh = (iota == far).astype(jnp.float32)
        cx = jnp.sum(X * oh, axis=-1, keepdims=True)
        cy = jnp.sum(Y * oh, axis=-1, keepdims=True)
        cz = jnp.sum(Z * oh, axis=-1, keepdims=True)
        slot = (iota_s == i).astype(jnp.float32)
        ax = ax + cx * slot
        ay = ay + cy * slot
        az = az + cz * slot
        dx = X - cx
        dy = Y - cy
        dz = Z - cz
        d = dx * dx + dy * dy + dz * dz
        dist = jnp.minimum(dist, d)
        m = jnp.max(dist, axis=-1, keepdims=True)
        far = jnp.min(jnp.where(dist == m, iota, N), axis=-1, keepdims=True)
        return dist, far, ax, ay, az

    init = (jnp.full((B, N), 1e10, jnp.float32),
            jnp.zeros((B, 1), jnp.int32),
            jnp.zeros((B, S), jnp.float32),
            jnp.zeros((B, S), jnp.float32),
            jnp.zeros((B, S), jnp.float32))
    _, _, ax, ay, az = lax.fori_loop(0, S, step, init)
    oref[0] = ax
    oref[1] = ay
    oref[2] = az


def _fps(xyzc, S):
    three, B, N = xyzc.shape
    fn = functools.partial(_fps_body, S, N)
    return pl.pallas_call(
        fn,
        grid=(1,),
        in_specs=[pl.BlockSpec((3, B, N), lambda i: (0, 0, 0))],
        out_specs=pl.BlockSpec((3, B, S), lambda i: (0, 0, 0)),
        out_shape=jax.ShapeDtypeStruct((3, B, S), jnp.float32),
    )(xyzc)


def _bq_body(N, r2, nsample, xref, cref, oref):
    x = xref[0]
    x3 = x[0:3, :]
    c = cref[0]
    TS = c.shape[0]
    aa = jnp.sum(c * c, axis=-1, keepdims=True)
    bb = jnp.sum(x3 * x3, axis=0, keepdims=True)
    ab = lax.dot_general(c, x3, (((1,), (0,)), ((), ())),
                         preferred_element_type=jnp.float32)
    sq = aa + bb - 2.0 * ab
    iota = lax.broadcasted_iota(jnp.int32, (TS, N), 1)
    midx = jnp.where(sq <= r2, iota, N)
    cols = []
    for _ in range(nsample):
        cur = jnp.min(midx, axis=-1, keepdims=True)
        cols.append(cur)
        midx = jnp.where(midx == cur, N, midx)
    g = jnp.concatenate(cols, axis=-1)
    g = jnp.where(g == N, cols[0], g)
    b = pl.program_id(0)
    oref[0] = g + b * N


def _ball_query(xyzT, new_xyz, radius, nsample):
    B, _, N = xyzT.shape
    S = new_xyz.shape[1]
    TS = min(S, 256)
    fn = functools.partial(_bq_body, N, radius * radius, nsample)
    return pl.pallas_call(
        fn,
        grid=(B, S // TS),
        in_specs=[
            pl.BlockSpec((1, 8, N), lambda b, s: (b, 0, 0)),
            pl.BlockSpec((1, TS, 3), lambda b, s: (b, s, 0)),
        ],
        out_specs=pl.BlockSpec((1, TS, nsample), lambda b, s: (b, s, 0)),
        out_shape=jax.ShapeDtypeStruct((B, S, nsample), jnp.int32),
    )(xyzT, new_xyz)


def _gather_rows(table, idx):
    from jax.experimental.pallas import tpu_sc as plsc
    V, D = table.shape
    Btot = idx.shape[0]
    info = plsc.get_sparse_core_info()
    NC, NS = info.num_cores, info.num_subcores
    NW = NC * NS
    R = Btot // NW
    chunk = R
    while chunk * D * 4 > 262144:
        chunk //= 2
    n_chunks = R // chunk
    mesh = plsc.VectorSubcoreMesh(core_axis_name="c", subcore_axis_name="s")

    @functools.partial(
        pl.kernel, mesh=mesh,
        out_type=jax.ShapeDtypeStruct((Btot, D), jnp.float32),
        scratch_types=[
            pltpu.VMEM((chunk,), jnp.int32),
            pltpu.VMEM((chunk, D), jnp.float32),
            pltpu.SemaphoreType.DMA,
        ],
    )
    def k(table_hbm, idx_hbm, out_hbm, idx_v, rows_v, sem):
        wid = lax.axis_index("s") * NC + lax.axis_index("c")
        base = wid * R
        for ci in range(n_chunks):
            off = base + ci * chunk
            pltpu.sync_copy(idx_hbm.at[pl.ds(off, chunk)], idx_v)
            pltpu.async_copy(table_hbm.at[idx_v], rows_v, sem).wait()
            pltpu.sync_copy(rows_v, out_hbm.at[pl.ds(off, chunk)])

    return k(table, idx)


def _sa_mlp_body(K, n_layers, gref, cref, *args):
    wrefs = args[:2 * n_layers]
    oref = args[2 * n_layers]
    g = gref[...]
    c = cref[...]
    TM = c.shape[0]
    Dp = g.shape[-1]
    delta = jnp.concatenate(
        [c, jnp.zeros((TM, Dp - 3), jnp.float32)], axis=-1)
    g = g - delta[None, :, :]
    x = g.reshape(K * TM, Dp)
    for li in range(n_layers):
        W = wrefs[2 * li][...]
        b = wrefs[2 * li + 1][...]
        x = jnp.dot(x, W, preferred_element_type=jnp.float32) + b
        x = jnp.maximum(x, 0.0)
    h = x.reshape(K, TM, x.shape[-1])
    oref[...] = jnp.max(h, axis=0)


def _sa_mlp(grouped, centers, layers):
    K, M, Dp = grouped.shape
    TM = min(M, 256)
    n_layers = len(layers)
    Dout = layers[-1][0].shape[1]
    wargs = []
    in_specs = [
        pl.BlockSpec((K, TM, Dp), lambda i: (0, i, 0)),
        pl.BlockSpec((TM, 3), lambda i: (i, 0)),
    ]
    for (W, b) in layers:
        if W.shape[0] != Dp:
            Wp = jnp.concatenate(
                [W, jnp.zeros((Dp - W.shape[0], W.shape[1]), W.dtype)], 0)
        else:
            Wp = W
        wargs += [Wp, b.reshape(1, -1)]
        in_specs += [
            pl.BlockSpec(Wp.shape, lambda i: (0, 0)),
            pl.BlockSpec((1, b.shape[0]), lambda i: (0, 0)),
        ]
        Dp = W.shape[1]
    fn = functools.partial(_sa_mlp_body, K, n_layers)
    return pl.pallas_call(
        fn,
        grid=(M // TM,),
        in_specs=in_specs,
        out_specs=pl.BlockSpec((TM, Dout), lambda i: (i, 0)),
        out_shape=jax.ShapeDtypeStruct((M, Dout), jnp.float32),
    )(grouped, centers, *wargs)


def _nn3_body(N2, xref, cref, iref, wref):
    x = xref[0]
    x3 = x[0:3, :]
    c = cref[0]
    TS = c.shape[0]
    aa = jnp.sum(c * c, axis=-1, keepdims=True)
    bb = jnp.sum(x3 * x3, axis=0, keepdims=True)
    ab = lax.dot_general(c, x3, (((1,), (0,)), ((), ())),
                         preferred_element_type=jnp.float32)
    sq = aa + bb - 2.0 * ab
    iota = lax.broadcasted_iota(jnp.int32, (TS, N2), 1)
    icols, dcols = [], []
    s = sq
    for _ in range(3):
        d = jnp.min(s, axis=-1, keepdims=True)
        i = jnp.min(jnp.where(s == d, iota, N2), axis=-1, keepdims=True)
        s = jnp.where(iota == i, jnp.float32(3.4e38), s)
        icols.append(i)
        dcols.append(d)
    dist = jnp.maximum(jnp.concatenate(dcols, axis=-1), 0.0)
    recip = 1.0 / (dist + 1e-8)
    w = recip / jnp.sum(recip, axis=-1, keepdims=True)
    b = pl.program_id(0)
    iref[0] = jnp.concatenate(icols, axis=-1) + b * N2
    wref[0] = w


def _nn3(xyz1, xyz2T):
    B, N1, _ = xyz1.shape
    N2 = xyz2T.shape[2]
    TS = min(N1, 256)
    fn = functools.partial(_nn3_body, N2)
    return pl.pallas_call(
        fn,
        grid=(B, N1 // TS),
        in_specs=[
            pl.BlockSpec((1, 8, N2), lambda b, s: (b, 0, 0)),
            pl.BlockSpec((1, TS, 3), lambda b, s: (b, s, 0)),
        ],
        out_specs=[
            pl.BlockSpec((1, TS, 3), lambda b, s: (b, s, 0)),
            pl.BlockSpec((1, TS, 3), lambda b, s: (b, s, 0)),
        ],
        out_shape=[
            jax.ShapeDtypeStruct((B, N1, 3), jnp.int32),
            jax.ShapeDtypeStruct((B, N1, 3), jnp.float32),
        ],
    )(xyz2T, xyz1)


def _fp_mlp_body(n_layers, gref, wref, fref, *args):
    wrefs = args[:2 * n_layers]
    oref = args[2 * n_layers]
    g = gref[...]
    w = wref[...]
    f1 = fref[...]
    interp = (g[0] * w[:, 0:1] + g[1] * w[:, 1:2]) + g[2] * w[:, 2:3]
    x = jnp.concatenate([f1, interp], axis=-1)
    for li in range(n_layers):
        W = wrefs[2 * li][...]
        b = wrefs[2 * li + 1][...]
        x = jnp.dot(x, W, preferred_element_type=jnp.float32) + b
        x = jnp.maximum(x, 0.0)
    oref[...] = x


def _fp_mlp(g, w, f1, layers):
    _, M, D2 = g.shape
    D1 = f1.shape[-1]
    TM = min(M, 512)
    n_layers = len(layers)
    Dout = layers[-1][0].shape[1]
    wargs = []
    in_specs = [
        pl.BlockSpec((3, TM, D2), lambda i: (0, i, 0)),
        pl.BlockSpec((TM, 3), lambda i: (i, 0)),
        pl.BlockSpec((TM, D1), lambda i: (i, 0)),
    ]
    for (W, b) in layers:
        wargs += [W, b.reshape(1, -1)]
        in_specs += [
            pl.BlockSpec(W.shape, lambda i: (0, 0)),
            pl.BlockSpec((1, b.shape[0]), lambda i: (0, 0)),
        ]
    fn = functools.partial(_fp_mlp_body, n_layers)
    return pl.pallas_call(
        fn,
        grid=(M // TM,),
        in_specs=in_specs,
        out_specs=pl.BlockSpec((TM, Dout), lambda i: (i, 0)),
        out_shape=jax.ShapeDtypeStruct((M, Dout), jnp.float32),
    )(g, w, f1, *wargs)


def _sa_level(xyz, feats, npoint, radius, nsample, layers):
    B, N, _ = xyz.shape
    C = feats.shape[-1]
    xyzc = jnp.transpose(xyz, (2, 0, 1))
    newc = _fps(xyzc, npoint)
    new_xyz = jnp.transpose(newc, (1, 2, 0))
    xyzT = jnp.concatenate(
        [jnp.transpose(xyz, (0, 2, 1)),
         jnp.zeros((B, 5, N), jnp.float32)], axis=1)
    gidx = _ball_query(xyzT, new_xyz, radius, nsample)
    Din = 3 + C
    Dp = ((Din + 127) // 128) * 128
    table = _pad_cols(jnp.concatenate([xyz, feats], -1),
                      Dp).reshape(B * N, Dp)
    M = B * npoint
    idx_flat = jnp.transpose(gidx.reshape(M, nsample)).reshape(-1)
    rows = _gather_rows(table, idx_flat)
    grouped = rows.reshape(nsample, M, Dp)
    centers = new_xyz.reshape(M, 3)
    feats_out = _sa_mlp(grouped, centers, layers).reshape(B, npoint, -1)
    return new_xyz, feats_out


def _fp_level(xyz1, xyz2, feats1, feats2, layers):
    B, N1, _ = xyz1.shape
    N2 = xyz2.shape[1]
    D2 = feats2.shape[-1]
    xyz2T = jnp.concatenate(
        [jnp.transpose(xyz2, (0, 2, 1)),
         jnp.zeros((B, 5, N2), jnp.float32)], axis=1)
    idx, w = _nn3(xyz1, xyz2T)
    M = B * N1
    table = feats2.reshape(B * N2, D2)
    idx_flat = jnp.transpose(idx.reshape(M, 3)).reshape(-1)
    rows = _gather_rows(table, idx_flat)
    g = rows.reshape(3, M, D2)
    out = _fp_mlp(g, w.reshape(M, 3), feats1.reshape(M, -1), layers)
    return out.reshape(B, N1, -1)


def kernel(pointcloud, params):
    xyz = pointcloud[..., :3]
    feats = pointcloud[..., 3:]
    l_xyz = [xyz]
    l_feats = [feats]
    for cfg, layers in zip(_SA_CFG, params["sa"]):
        nx, nf = _sa_level(l_xyz[-1], l_feats[-1], cfg[0], cfg[1], cfg[2],
                           layers)
        l_xyz.append(nx)
        l_feats.append(nf)
    for i in range(-1, -5, -1):
        l_feats[i - 1] = _fp_level(l_xyz[i - 1], l_xyz[i], l_feats[i - 1],
                                   l_feats[i], params["fp"][i])
    return l_feats[0]
